# Initial kernel scaffold; baseline (speedup 1.0000x reference)
import functools

import jax
import jax.numpy as jnp
from jax import lax
from jax.experimental import pallas as pl
from jax.experimental.pallas import tpu as pltpu
from jax.experimental.pallas import tpu_sc as plsc

B, F, V, D = 16384, 26, 100000, 32
IN_DIM = F * D
EPS = 1e-5
INV = 1.0 / (1.0 + EPS) ** 0.5

NC, NS = 2, 16
NW = NC * NS
N = B * F
ROWS_PER_W = N // NW
CHUNK = 1664
NCHUNK = ROWS_PER_W // CHUNK


def _gather_body(idx_hbm, table_hbm, out_hbm, idx_v, rows_v, sem):
    wid = lax.axis_index("s") * NC + lax.axis_index("c")
    base = wid * ROWS_PER_W

    def step(i, carry):
        off = base + i * CHUNK
        pltpu.sync_copy(idx_hbm.at[pl.ds(off, CHUNK)], idx_v)
        pltpu.async_copy(table_hbm.at[idx_v], rows_v, sem).wait()
        pltpu.sync_copy(rows_v, out_hbm.at[pl.ds(off, CHUNK)])
        return carry

    lax.fori_loop(0, NCHUNK, step, 0)


_sc_gather = functools.partial(
    pl.kernel,
    out_type=jax.ShapeDtypeStruct((N, D), jnp.float32),
    mesh=plsc.VectorSubcoreMesh(
        core_axis_name="c", subcore_axis_name="s", num_cores=NC, num_subcores=NS
    ),
    scratch_types=[
        pltpu.VMEM((CHUNK,), jnp.int32),
        pltpu.VMEM((CHUNK, D), jnp.float32),
        pltpu.SemaphoreType.DMA,
    ],
)(_gather_body)


def _mlp_body(h_ref, w0, b0, g0, be0, w1, b1, g1, be1, w2, b2, g2, be2, wo, bo,
              out_ref):
    h = h_ref[...]
    z = jnp.dot(h, w0[...], preferred_element_type=jnp.float32) + b0[...]
    z = jnp.maximum(z, 0.0) * (g0[...] * INV) + be0[...]
    z = jnp.dot(z, w1[...], preferred_element_type=jnp.float32) + b1[...]
    z = jnp.maximum(z, 0.0) * (g1[...] * INV) + be1[...]
    z = jnp.dot(z, w2[...], preferred_element_type=jnp.float32) + b2[...]
    z = jnp.maximum(z, 0.0) * (g2[...] * INV) + be2[...]
    o = jnp.dot(z, wo[...], preferred_element_type=jnp.float32) + bo[...]
    out_ref[...] = jax.nn.sigmoid(o)


BT = 1024


def _mlp(h, W0T, b0, g0, be0, W1T, b1, g1, be1, W2T, b2, g2, be2, WoT, bout):
    full = lambda shape: pl.BlockSpec(shape, lambda i: (0,) * len(shape))
    return pl.pallas_call(
        _mlp_body,
        grid=(B // BT,),
        in_specs=[
            pl.BlockSpec((BT, IN_DIM), lambda i: (i, 0)),
            full(W0T.shape), full(b0.shape), full(g0.shape), full(be0.shape),
            full(W1T.shape), full(b1.shape), full(g1.shape), full(be1.shape),
            full(W2T.shape), full(b2.shape), full(g2.shape), full(be2.shape),
            full(WoT.shape), full(bout.shape),
        ],
        out_specs=pl.BlockSpec((BT, 1), lambda i: (i, 0)),
        out_shape=jax.ShapeDtypeStruct((B, 1), jnp.float32),
    )(h, W0T, b0, g0, be0, W1T, b1, g1, be1, W2T, b2, g2, be2, WoT, bout)


def kernel(x, emb_tables, W0, b0, g0, be0, W1, b1, g1, be1, W2, b2, g2, be2,
           Wout, bout):
    flat_idx = (x + jnp.arange(F, dtype=jnp.int32)[None, :] * V).reshape(N)
    table = emb_tables.reshape(F * V, D)
    rows = _sc_gather(flat_idx, table)
    h = rows.reshape(B, IN_DIM)
    return _mlp(h, W0.T, b0, g0, be0, W1.T, b1, g1, be1, W2.T, b2, g2, be2,
                Wout.T, bout)


# speedup vs baseline: 7.8708x; 7.8708x over previous
import functools

import jax
import jax.numpy as jnp
from jax import lax
from jax.experimental import pallas as pl
from jax.experimental.pallas import tpu as pltpu
from jax.experimental.pallas import tpu_sc as plsc

B, F, V, D = 16384, 26, 100000, 32
IN_DIM = F * D
EPS = 1e-5
INV = 1.0 / (1.0 + EPS) ** 0.5

NC, NS = 2, 16
NW = NC * NS
N = B * F
ROWS_PER_W = N // NW
CHUNK = 1664
NCHUNK = ROWS_PER_W // CHUNK


def _gather_body(idx_hbm, table_hbm, out_hbm, idx_v, rows_v, sem):
    wid = lax.axis_index("s") * NC + lax.axis_index("c")
    base = wid * ROWS_PER_W

    def step(i, carry):
        off = base + i * CHUNK
        pltpu.sync_copy(idx_hbm.at[pl.ds(off, CHUNK)], idx_v)
        pltpu.async_copy(table_hbm.at[idx_v], rows_v, sem).wait()
        pltpu.sync_copy(rows_v, out_hbm.at[pl.ds(off, CHUNK)])
        return carry

    lax.fori_loop(0, NCHUNK, step, 0)


@functools.cache
def _sc_gather():
    return pl.kernel(
        _gather_body,
        out_type=jax.ShapeDtypeStruct((N, D), jnp.float32),
        mesh=plsc.VectorSubcoreMesh(
            core_axis_name="c", subcore_axis_name="s",
            num_cores=NC, num_subcores=NS,
        ),
        scratch_types=[
            pltpu.VMEM((CHUNK,), jnp.int32),
            pltpu.VMEM((CHUNK, D), jnp.float32),
            pltpu.SemaphoreType.DMA,
        ],
        compiler_params=pltpu.CompilerParams(use_tc_tiling_on_sc=False),
    )


def _mlp_body(h_ref, w0, b0, g0, be0, w1, b1, g1, be1, w2, b2, g2, be2, wo, bo,
              out_ref):
    h = h_ref[...]
    z = jnp.dot(h, w0[...], preferred_element_type=jnp.float32) + b0[...]
    z = jnp.maximum(z, 0.0) * (g0[...] * INV) + be0[...]
    z = jnp.dot(z, w1[...], preferred_element_type=jnp.float32) + b1[...]
    z = jnp.maximum(z, 0.0) * (g1[...] * INV) + be1[...]
    z = jnp.dot(z, w2[...], preferred_element_type=jnp.float32) + b2[...]
    z = jnp.maximum(z, 0.0) * (g2[...] * INV) + be2[...]
    o = jnp.dot(z, wo[...], preferred_element_type=jnp.float32) + bo[...]
    out_ref[...] = jax.nn.sigmoid(o)


BT = 1024


def _mlp(h, W0T, b0, g0, be0, W1T, b1, g1, be1, W2T, b2, g2, be2, WoT, bout):
    full = lambda shape: pl.BlockSpec(shape, lambda i: (0,) * len(shape))
    return pl.pallas_call(
        _mlp_body,
        grid=(B // BT,),
        in_specs=[
            pl.BlockSpec((BT, IN_DIM), lambda i: (i, 0)),
            full(W0T.shape), full(b0.shape), full(g0.shape), full(be0.shape),
            full(W1T.shape), full(b1.shape), full(g1.shape), full(be1.shape),
            full(W2T.shape), full(b2.shape), full(g2.shape), full(be2.shape),
            full(WoT.shape), full(bout.shape),
        ],
        out_specs=pl.BlockSpec((BT, 1), lambda i: (i, 0)),
        out_shape=jax.ShapeDtypeStruct((B, 1), jnp.float32),
    )(h, W0T, b0, g0, be0, W1T, b1, g1, be1, W2T, b2, g2, be2, WoT, bout)


def kernel(x, emb_tables, W0, b0, g0, be0, W1, b1, g1, be1, W2, b2, g2, be2,
           Wout, bout):
    flat_idx = (x + jnp.arange(F, dtype=jnp.int32)[None, :] * V).reshape(N)
    table = emb_tables.reshape(F * V, D)
    rows = _sc_gather()(flat_idx, table)
    h = rows.reshape(B, IN_DIM)
    return _mlp(h, W0.T, b0, g0, be0, W1.T, b1, g1, be1, W2.T, b2, g2, be2,
                Wout.T, bout)
